# baseline (speedup 1.0000x reference)
import jax
import jax.numpy as jnp
from jax.experimental import pallas as pl

N_TOK = 18432
K_CODES = 8192
DIM = 64
BN = 256


def _argmin_body(zsq_ref, z_ref, cbt_ref, csq_ref, idx_ref):
    mm = jax.lax.dot_general(
        z_ref[...], cbt_ref[...], (((1,), (0,)), ((), ())),
        preferred_element_type=jnp.float32)
    d2 = (zsq_ref[...] + csq_ref[...]) - 2.0 * mm
    d = jnp.maximum(d2, 0.0)
    s = jnp.where(d == 0.0, 0.0, d * jax.lax.rsqrt(d))
    mv = jnp.min(s, axis=1, keepdims=True)
    iota = jax.lax.broadcasted_iota(jnp.int32, s.shape, 1)
    idx_ref[...] = jnp.min(jnp.where(s == mv, iota, K_CODES), axis=1)


def _row_sumsq(x):
    q = x * x
    cs = []
    for s in range(8):
        acc = q[:, s]
        for k in range(1, 8):
            acc = q[:, 8 * k + s] + acc
        cs.append(acc)
    t = [cs[0] + cs[4], cs[1] + cs[5], cs[2] + cs[6], cs[3] + cs[7]]
    u0 = t[0] + t[2]
    u1 = t[1] + t[3]
    return u1 + u0


def _nearest_code(z, codebook):
    z_sq = _row_sumsq(z)[:, None]
    c_sq = _row_sumsq(codebook)[None, :]
    cb_t = codebook.T
    return pl.pallas_call(
        _argmin_body,
        grid=(N_TOK // BN,),
        in_specs=[
            pl.BlockSpec((BN, 1), lambda i: (i, 0)),
            pl.BlockSpec((BN, DIM), lambda i: (i, 0)),
            pl.BlockSpec((DIM, K_CODES), lambda i: (0, 0)),
            pl.BlockSpec((1, K_CODES), lambda i: (0, 0)),
        ],
        out_specs=pl.BlockSpec((BN,), lambda i: (i,)),
        out_shape=jax.ShapeDtypeStruct((N_TOK,), jnp.int32),
    )(z_sq, z, cb_t, c_sq)


def kernel(z, codebook):
    indices = _nearest_code(z, codebook)
    z_q = jnp.take(codebook, indices, axis=0)
    commitment_loss = jnp.mean((z_q - z) ** 2) * 0.25
    z_q_st = z + (z_q - z)
    residual = z - z_q_st
    counts = jnp.bincount(indices, length=K_CODES).astype(jnp.float32)
    avg_probs = counts / jnp.sum(counts)
    perplexity = jnp.exp(-jnp.sum(avg_probs * jnp.log(avg_probs + 1e-10)))
    return (z_q_st, residual, indices, commitment_loss, perplexity)
